# baseline (speedup 1.0000x reference)
import jax
import jax.numpy as jnp
from jax.experimental import pallas as pl

B = 4
N_P = 8192
D = 768
PB = 256


def _add_kernel(patches_ref, pos_ref, out_ref):
    out_ref[...] = patches_ref[...] + pos_ref[...][None, :, :]


def kernel(patches, pos_table):
    grid = (N_P // PB,)
    return pl.pallas_call(
        _add_kernel,
        grid=grid,
        in_specs=[
            pl.BlockSpec((B, PB, D), lambda i: (0, i, 0)),
            pl.BlockSpec((PB, D), lambda i: (i, 0)),
        ],
        out_specs=pl.BlockSpec((B, PB, D), lambda i: (0, i, 0)),
        out_shape=jax.ShapeDtypeStruct((B, N_P, D), jnp.float32),
    )(patches, pos_table)
